# baseline (speedup 1.0000x reference)
import functools

import jax
import jax.numpy as jnp
from jax.experimental import pallas as pl
from jax.experimental.pallas import tpu as pltpu

_WEIGHT = 0.01


def _chamfer_body(x_ref, ys_ref, x2_ref, y2_ref, o_ref, rowacc_ref,
                  colmin_ref, *, n_blocks, m_blocks, tile_m, n, m):
    nb = pl.program_id(1)
    mb = pl.program_id(2)

    x = x_ref[0]
    x2 = x2_ref[0, 0, :]
    y2 = y2_ref[0, 0, :]

    chunk = 256
    gm = None
    bcol_parts = []
    for c in range(tile_m // chunk):
        sl_c = slice(c * chunk, (c + 1) * chunk)
        ys_c = ys_ref[0, sl_c, :]
        t = jax.lax.dot_general(
            x, ys_c, (((1,), (1,)), ((), ())),
            preferred_element_type=jnp.float32)

        bcol_parts.append(jnp.min(t + x2[:, None], axis=0))

        y2_c = y2[sl_c]
        for g in range(chunk // 128):
            sl_g = slice(g * 128, (g + 1) * 128)
            part = t[:, sl_g] + y2_c[sl_g][None, :]
            gm = part if gm is None else jnp.minimum(gm, part)
    bcol = jnp.concatenate(bcol_parts)

    @pl.when(jnp.logical_and(nb == 0, mb == 0))
    def _():
        o_ref[0, 0, :] = jnp.zeros((128,), jnp.float32)

    @pl.when(mb == 0)
    def _():
        rowacc_ref[:, :] = gm

    @pl.when(mb > 0)
    def _():
        rowacc_ref[:, :] = jnp.minimum(rowacc_ref[:, :], gm)

    @pl.when(mb == m_blocks - 1)
    def _():
        rowmin = jnp.min(rowacc_ref[:, :], axis=1)
        cham_x = jnp.maximum(rowmin + x2, 0.0)
        o_ref[0, 0, :] += jnp.full((128,), jnp.sum(cham_x) * (1.0 / n))

    sl = pl.ds(mb * tile_m, tile_m)

    @pl.when(nb == 0)
    def _():
        colmin_ref[0, sl] = bcol

    @pl.when(nb > 0)
    def _():
        colmin_ref[0, sl] = jnp.minimum(colmin_ref[0, sl], bcol)

    @pl.when(nb == n_blocks - 1)
    def _():
        cham_y = jnp.maximum(colmin_ref[0, sl] + y2, 0.0)
        o_ref[0, 0, :] += jnp.full((128,), jnp.sum(cham_y) * (1.0 / m))


def kernel(transformed_source, transformed_target):
    xf = transformed_source.astype(jnp.float32)
    yf = transformed_target.astype(jnp.float32)
    b, n, d = xf.shape
    _, m, _ = yf.shape
    x = xf.astype(jnp.bfloat16)
    ys = (-2.0 * yf).astype(jnp.bfloat16)
    x32 = x.astype(jnp.float32)
    ys32 = ys.astype(jnp.float32)
    x2 = jnp.sum(x32 * x32, axis=-1)[:, None, :]
    y2 = 0.25 * jnp.sum(ys32 * ys32, axis=-1)[:, None, :]

    tile_n = 4096
    tile_m = 4096
    n_blocks = n // tile_n
    m_blocks = m // tile_m

    body = functools.partial(
        _chamfer_body, n_blocks=n_blocks, m_blocks=m_blocks, tile_m=tile_m,
        n=n, m=m)

    out = pl.pallas_call(
        body,
        grid=(b, n_blocks, m_blocks),
        compiler_params=pltpu.CompilerParams(
            dimension_semantics=("parallel", "arbitrary", "arbitrary")),
        in_specs=[
            pl.BlockSpec((1, tile_n, d), lambda bi, ni, mi: (bi, ni, 0)),
            pl.BlockSpec((1, tile_m, d), lambda bi, ni, mi: (bi, mi, 0)),
            pl.BlockSpec((1, 1, tile_n), lambda bi, ni, mi: (bi, 0, ni)),
            pl.BlockSpec((1, 1, tile_m), lambda bi, ni, mi: (bi, 0, mi)),
        ],
        out_specs=pl.BlockSpec((1, 1, 128), lambda bi, ni, mi: (bi, 0, 0)),
        out_shape=jax.ShapeDtypeStruct((b, 1, 128), jnp.float32),
        scratch_shapes=[
            pltpu.VMEM((tile_n, 128), jnp.float32),
            pltpu.VMEM((1, m), jnp.float32),
        ],
    )(x, ys, x2, y2)

    return _WEIGHT * jnp.mean(out[:, 0, 0])


# speedup vs baseline: 1.1214x; 1.1214x over previous
import functools

import jax
import jax.numpy as jnp
from jax.experimental import pallas as pl
from jax.experimental.pallas import tpu as pltpu

_WEIGHT = 0.01


def _chamfer_body(x_ref, ys_ref, x2_ref, y2_ref, o_ref, rowacc_ref,
                  colmin_ref, *, n_blocks, m_blocks, tile_m, n, m):
    nb = pl.program_id(1)
    mb = pl.program_id(2)

    x = x_ref[0]
    x2 = x2_ref[0, 0, :]
    y2 = y2_ref[0, 0, :]

    chunk = 256
    gm = None
    bcol_parts = []
    for c in range(tile_m // chunk):
        sl_c = slice(c * chunk, (c + 1) * chunk)
        ys_c = ys_ref[0, sl_c, :]
        t = jax.lax.dot_general(
            x, ys_c, (((1,), (1,)), ((), ())),
            preferred_element_type=jnp.float32)

        bcol_parts.append(jnp.min(t + x2[:, None], axis=0))

        y2_c = y2[sl_c]
        for g in range(chunk // 128):
            sl_g = slice(g * 128, (g + 1) * 128)
            part = t[:, sl_g] + y2_c[sl_g][None, :]
            gm = part if gm is None else jnp.minimum(gm, part)
    bcol = jnp.concatenate(bcol_parts)

    @pl.when(jnp.logical_and(nb == 0, mb == 0))
    def _():
        o_ref[0, 0, :] = jnp.zeros((128,), jnp.float32)

    @pl.when(mb == 0)
    def _():
        rowacc_ref[:, :] = gm

    @pl.when(mb > 0)
    def _():
        rowacc_ref[:, :] = jnp.minimum(rowacc_ref[:, :], gm)

    @pl.when(mb == m_blocks - 1)
    def _():
        rowmin = jnp.min(rowacc_ref[:, :], axis=1)
        cham_x = jnp.maximum(rowmin + x2, 0.0)
        o_ref[0, 0, :] += jnp.full((128,), jnp.sum(cham_x) * (1.0 / n))

    sl = pl.ds(mb * tile_m, tile_m)

    @pl.when(nb == 0)
    def _():
        colmin_ref[0, sl] = bcol

    @pl.when(nb > 0)
    def _():
        colmin_ref[0, sl] = jnp.minimum(colmin_ref[0, sl], bcol)

    @pl.when(nb == n_blocks - 1)
    def _():
        cham_y = jnp.maximum(colmin_ref[0, sl] + y2, 0.0)
        o_ref[0, 0, :] += jnp.full((128,), jnp.sum(cham_y) * (1.0 / m))


def kernel(transformed_source, transformed_target):
    xf = transformed_source.astype(jnp.float32)
    yf = transformed_target.astype(jnp.float32)
    b, n, d = xf.shape
    _, m, _ = yf.shape
    x = xf.astype(jnp.bfloat16)
    ys = (-2.0 * yf).astype(jnp.bfloat16)
    x32 = x.astype(jnp.float32)
    ys32 = ys.astype(jnp.float32)
    x2 = jnp.sum(x32 * x32, axis=-1)[:, None, :]
    y2 = 0.25 * jnp.sum(ys32 * ys32, axis=-1)[:, None, :]

    tile_n = 4096
    tile_m = 1024
    n_blocks = n // tile_n
    m_blocks = m // tile_m

    body = functools.partial(
        _chamfer_body, n_blocks=n_blocks, m_blocks=m_blocks, tile_m=tile_m,
        n=n, m=m)

    out = pl.pallas_call(
        body,
        grid=(b, n_blocks, m_blocks),
        compiler_params=pltpu.CompilerParams(
            dimension_semantics=("parallel", "arbitrary", "arbitrary")),
        in_specs=[
            pl.BlockSpec((1, tile_n, d), lambda bi, ni, mi: (bi, ni, 0)),
            pl.BlockSpec((1, tile_m, d), lambda bi, ni, mi: (bi, mi, 0)),
            pl.BlockSpec((1, 1, tile_n), lambda bi, ni, mi: (bi, 0, ni)),
            pl.BlockSpec((1, 1, tile_m), lambda bi, ni, mi: (bi, 0, mi)),
        ],
        out_specs=pl.BlockSpec((1, 1, 128), lambda bi, ni, mi: (bi, 0, 0)),
        out_shape=jax.ShapeDtypeStruct((b, 1, 128), jnp.float32),
        scratch_shapes=[
            pltpu.VMEM((tile_n, 128), jnp.float32),
            pltpu.VMEM((1, m), jnp.float32),
        ],
    )(x, ys, x2, y2)

    return _WEIGHT * jnp.mean(out[:, 0, 0])


# speedup vs baseline: 1.2044x; 1.0740x over previous
import functools

import jax
import jax.numpy as jnp
from jax.experimental import pallas as pl
from jax.experimental.pallas import tpu as pltpu

_WEIGHT = 0.01


def _chamfer_body(x_ref, ys_ref, x2_ref, y2_ref, o_ref, rowacc_ref,
                  colmin_ref, *, n_blocks, m_blocks, tile_m, n, m):
    nb = pl.program_id(1)
    mb = pl.program_id(2)

    x = x_ref[0]
    x2 = x2_ref[0, 0, :]
    y2 = y2_ref[0, 0, :]

    chunk = 256
    gm = None
    for c in range(tile_m // chunk):
        sl_c = slice(c * chunk, (c + 1) * chunk)
        ys_c = ys_ref[0, sl_c, :]
        t = jax.lax.dot_general(
            x, ys_c, (((1,), (1,)), ((), ())),
            preferred_element_type=jnp.float32)

        bc = jnp.min(t + x2[:, None], axis=0)
        sl_w = pl.ds(mb * tile_m + c * chunk, chunk)

        @pl.when(nb == 0)
        def _():
            colmin_ref[0, sl_w] = bc

        @pl.when(nb > 0)
        def _():
            colmin_ref[0, sl_w] = jnp.minimum(colmin_ref[0, sl_w], bc)

        y2_c = y2[sl_c]
        for g in range(chunk // 128):
            sl_g = slice(g * 128, (g + 1) * 128)
            part = t[:, sl_g] + y2_c[sl_g][None, :]
            gm = part if gm is None else jnp.minimum(gm, part)

    @pl.when(jnp.logical_and(nb == 0, mb == 0))
    def _():
        o_ref[0, 0, :] = jnp.zeros((128,), jnp.float32)

    @pl.when(mb == 0)
    def _():
        rowacc_ref[:, :] = gm

    @pl.when(mb > 0)
    def _():
        rowacc_ref[:, :] = jnp.minimum(rowacc_ref[:, :], gm)

    @pl.when(mb == m_blocks - 1)
    def _():
        rowmin = jnp.min(rowacc_ref[:, :], axis=1)
        cham_x = jnp.maximum(rowmin + x2, 0.0)
        o_ref[0, 0, :] += jnp.full((128,), jnp.sum(cham_x) * (1.0 / n))

    @pl.when(nb == n_blocks - 1)
    def _():
        sl = pl.ds(mb * tile_m, tile_m)
        cham_y = jnp.maximum(colmin_ref[0, sl] + y2, 0.0)
        o_ref[0, 0, :] += jnp.full((128,), jnp.sum(cham_y) * (1.0 / m))


def kernel(transformed_source, transformed_target):
    xf = transformed_source.astype(jnp.float32)
    yf = transformed_target.astype(jnp.float32)
    b, n, d = xf.shape
    _, m, _ = yf.shape
    x = xf.astype(jnp.bfloat16)
    ys = (-2.0 * yf).astype(jnp.bfloat16)
    x32 = x.astype(jnp.float32)
    ys32 = ys.astype(jnp.float32)
    x2 = jnp.sum(x32 * x32, axis=-1)[:, None, :]
    y2 = 0.25 * jnp.sum(ys32 * ys32, axis=-1)[:, None, :]

    tile_n = 4096
    tile_m = 2048
    n_blocks = n // tile_n
    m_blocks = m // tile_m

    body = functools.partial(
        _chamfer_body, n_blocks=n_blocks, m_blocks=m_blocks, tile_m=tile_m,
        n=n, m=m)

    out = pl.pallas_call(
        body,
        grid=(b, n_blocks, m_blocks),
        compiler_params=pltpu.CompilerParams(
            dimension_semantics=("parallel", "arbitrary", "arbitrary")),
        in_specs=[
            pl.BlockSpec((1, tile_n, d), lambda bi, ni, mi: (bi, ni, 0)),
            pl.BlockSpec((1, tile_m, d), lambda bi, ni, mi: (bi, mi, 0)),
            pl.BlockSpec((1, 1, tile_n), lambda bi, ni, mi: (bi, 0, ni)),
            pl.BlockSpec((1, 1, tile_m), lambda bi, ni, mi: (bi, 0, mi)),
        ],
        out_specs=pl.BlockSpec((1, 1, 128), lambda bi, ni, mi: (bi, 0, 0)),
        out_shape=jax.ShapeDtypeStruct((b, 1, 128), jnp.float32),
        scratch_shapes=[
            pltpu.VMEM((tile_n, 128), jnp.float32),
            pltpu.VMEM((1, m), jnp.float32),
        ],
    )(x, ys, x2, y2)

    return _WEIGHT * jnp.mean(out[:, 0, 0])


# speedup vs baseline: 1.2182x; 1.0115x over previous
import functools

import jax
import jax.numpy as jnp
from jax.experimental import pallas as pl
from jax.experimental.pallas import tpu as pltpu

_WEIGHT = 0.01


def _chamfer_body(x_ref, ys_ref, x2_ref, y2_ref, o_ref, rowacc_ref,
                  colmin_ref, *, n_blocks, m_blocks, tile_m, n, m):
    nb = pl.program_id(1)
    mb = pl.program_id(2)

    x = x_ref[0]
    x2 = x2_ref[0, 0, :]
    y2 = y2_ref[0, 0, :]

    chunk = 256
    gm = None
    bcol_parts = []
    for c in range(tile_m // chunk):
        sl_c = slice(c * chunk, (c + 1) * chunk)
        ys_c = ys_ref[0, sl_c, :]
        t = jax.lax.dot_general(
            x, ys_c, (((1,), (1,)), ((), ())),
            preferred_element_type=jnp.float32)

        bcol_parts.append(jnp.min(t + x2[:, None], axis=0))

        y2_c = y2[sl_c]
        for g in range(chunk // 128):
            sl_g = slice(g * 128, (g + 1) * 128)
            part = t[:, sl_g] + y2_c[sl_g][None, :]
            gm = part if gm is None else jnp.minimum(gm, part)
    bcol = jnp.concatenate(bcol_parts)

    @pl.when(jnp.logical_and(nb == 0, mb == 0))
    def _():
        o_ref[0, 0, :] = jnp.zeros((128,), jnp.float32)

    @pl.when(mb == 0)
    def _():
        rowacc_ref[:, :] = gm

    @pl.when(mb > 0)
    def _():
        rowacc_ref[:, :] = jnp.minimum(rowacc_ref[:, :], gm)

    @pl.when(mb == m_blocks - 1)
    def _():
        rowmin = jnp.min(rowacc_ref[:, :], axis=1)
        cham_x = jnp.maximum(rowmin + x2, 0.0)
        o_ref[0, 0, :] += jnp.full((128,), jnp.sum(cham_x) * (1.0 / n))

    sl = pl.ds(mb * tile_m, tile_m)

    @pl.when(nb == 0)
    def _():
        colmin_ref[0, sl] = bcol

    @pl.when(nb > 0)
    def _():
        colmin_ref[0, sl] = jnp.minimum(colmin_ref[0, sl], bcol)

    @pl.when(nb == n_blocks - 1)
    def _():
        cham_y = jnp.maximum(colmin_ref[0, sl] + y2, 0.0)
        o_ref[0, 0, :] += jnp.full((128,), jnp.sum(cham_y) * (1.0 / m))


def kernel(transformed_source, transformed_target):
    xf = transformed_source.astype(jnp.float32)
    yf = transformed_target.astype(jnp.float32)
    b, n, d = xf.shape
    _, m, _ = yf.shape
    x = xf.astype(jnp.bfloat16)
    ys = (-2.0 * yf).astype(jnp.bfloat16)
    x32 = x.astype(jnp.float32)
    ys32 = ys.astype(jnp.float32)
    x2 = jnp.sum(x32 * x32, axis=-1)[:, None, :]
    y2 = 0.25 * jnp.sum(ys32 * ys32, axis=-1)[:, None, :]

    tile_n = 4096
    tile_m = 2048
    n_blocks = n // tile_n
    m_blocks = m // tile_m

    body = functools.partial(
        _chamfer_body, n_blocks=n_blocks, m_blocks=m_blocks, tile_m=tile_m,
        n=n, m=m)

    out = pl.pallas_call(
        body,
        grid=(b, n_blocks, m_blocks),
        compiler_params=pltpu.CompilerParams(
            dimension_semantics=("parallel", "arbitrary", "arbitrary")),
        in_specs=[
            pl.BlockSpec((1, tile_n, d), lambda bi, ni, mi: (bi, ni, 0)),
            pl.BlockSpec((1, tile_m, d), lambda bi, ni, mi: (bi, mi, 0)),
            pl.BlockSpec((1, 1, tile_n), lambda bi, ni, mi: (bi, 0, ni)),
            pl.BlockSpec((1, 1, tile_m), lambda bi, ni, mi: (bi, 0, mi)),
        ],
        out_specs=pl.BlockSpec((1, 1, 128), lambda bi, ni, mi: (bi, 0, 0)),
        out_shape=jax.ShapeDtypeStruct((b, 1, 128), jnp.float32),
        scratch_shapes=[
            pltpu.VMEM((tile_n, 128), jnp.float32),
            pltpu.VMEM((1, m), jnp.float32),
        ],
    )(x, ys, x2, y2)

    return _WEIGHT * jnp.mean(out[:, 0, 0])
